# Initial kernel scaffold; baseline (speedup 1.0000x reference)
import jax
import jax.numpy as jnp
from jax.experimental import pallas as pl


def kernel(cls_token, router_w, router_b, list_indices):
    raise NotImplementedError("write your pallas kernel here")


# speedup vs baseline: 3.9411x; 3.9411x over previous
import functools

import jax
import jax.numpy as jnp
from jax import lax
from jax.experimental import pallas as pl
from jax.experimental.pallas import tpu as pltpu
from jax.experimental.pallas import tpu_sc as plsc

DIM = 1024
NUM_EXPERTS = 16
N_FRQ = 3000
TOPK = 8
BATCH = 64
NN = DIM * DIM

NW = 32
ROWS_PER_W = BATCH // NW
SEG = ROWS_PER_W * NN
IDX_PER_W = ROWS_PER_W * TOPK * N_FRQ
CW = 128
NCHUNK = IDX_PER_W // CW
ZW = 65536
NZCOPY = SEG // ZW


def _route_body(cls_ref, rw_ref, rb_ref, li_ref, idx_ref):
    logits = lax.dot_general(
        cls_ref[...], rw_ref[...], (((1,), (1,)), ((), ())),
        preferred_element_type=jnp.float32,
    ) + rb_ref[...][None, :]
    m = jnp.max(logits, axis=1, keepdims=True)
    e = jnp.exp(logits - m)
    probs = e / jnp.sum(e, axis=1, keepdims=True)

    iota_e = lax.broadcasted_iota(jnp.int32, (BATCH, NUM_EXPERTS), 1)
    work = probs
    experts = []
    for _ in range(TOPK):
        mx = jnp.max(work, axis=1, keepdims=True)
        cand = jnp.where(work == mx, iota_e, NUM_EXPERTS)
        ek = jnp.min(cand, axis=1, keepdims=True)
        experts.append(ek)
        work = jnp.where(iota_e == ek, -jnp.inf, work)
    exp_idx = jnp.concatenate(experts, axis=1)

    li_f = li_ref[...].astype(jnp.float32)
    onehot_iota = lax.broadcasted_iota(jnp.int32, (BATCH, NUM_EXPERTS), 1)
    row_off = lax.broadcasted_iota(jnp.int32, (BATCH, N_FRQ), 0) * NN
    for k in range(TOPK):
        onehot = (onehot_iota == exp_idx[:, k:k + 1]).astype(jnp.float32)
        sel = lax.dot_general(
            onehot, li_f, (((1,), (0,)), ((), ())),
            preferred_element_type=jnp.float32,
            precision=lax.Precision.HIGHEST,
        )
        idx_ref[:, pl.ds(k * N_FRQ, N_FRQ)] = sel.astype(jnp.int32) + row_off


def _route(cls_token, router_w, router_b, li):
    return pl.pallas_call(
        _route_body,
        out_shape=jax.ShapeDtypeStruct((BATCH, TOPK * N_FRQ), jnp.int32),
    )(cls_token, router_w, router_b, li)


def _sc_body(idx_hbm, out_hbm, zeros_v, ones_v, idx_v, zsem, ssem):
    wid = lax.axis_index("s") * 2 + lax.axis_index("c")

    def fill_z(i, _):
        zeros_v[pl.ds(i * 16, 16)] = jnp.zeros((16,), jnp.float32)
        return ()
    lax.fori_loop(0, ZW // 16, fill_z, (), unroll=8)
    for i in range(CW // 16):
        ones_v[pl.ds(i * 16, 16)] = jnp.ones((16,), jnp.float32)

    pltpu.sync_copy(idx_hbm.at[wid], idx_v)

    base = wid * SEG

    def zero_group(g, _):
        copies = [
            pltpu.async_copy(
                zeros_v, out_hbm.at[pl.ds(base + (g * 8 + i) * ZW, ZW)], zsem)
            for i in range(8)
        ]
        for c in copies:
            c.wait()
        return ()
    lax.fori_loop(0, NZCOPY // 8, zero_group, ())

    def scat_group(g, _):
        copies = [
            pltpu.async_copy(ones_v, out_hbm.at[idx_v.at[g * 8 + i]], ssem)
            for i in range(8)
        ]
        for c in copies:
            c.wait()
        return ()
    ngroups = NCHUNK // 8
    lax.fori_loop(0, ngroups, scat_group, ())
    tail = [
        pltpu.async_copy(ones_v, out_hbm.at[idx_v.at[ngroups * 8 + i]], ssem)
        for i in range(NCHUNK - ngroups * 8)
    ]
    for c in tail:
        c.wait()


@functools.partial(
    pl.kernel,
    out_type=jax.ShapeDtypeStruct((BATCH * NN,), jnp.float32),
    mesh=plsc.VectorSubcoreMesh(core_axis_name="c", subcore_axis_name="s",
                                num_cores=2, num_subcores=16),
    scratch_types=[
        pltpu.VMEM((ZW,), jnp.float32),
        pltpu.VMEM((CW,), jnp.float32),
        pltpu.VMEM((NCHUNK, CW), jnp.int32),
        pltpu.SemaphoreType.DMA,
        pltpu.SemaphoreType.DMA,
    ],
)
def _sc_scatter(idx_hbm, out_hbm, zeros_v, ones_v, idx_v, zsem, ssem):
    _sc_body(idx_hbm, out_hbm, zeros_v, ones_v, idx_v, zsem, ssem)


def kernel(cls_token, router_w, router_b, list_indices):
    li = list_indices.astype(jnp.int32)
    idx = _route(cls_token, router_w, router_b, li)
    idx3 = idx.reshape(NW, NCHUNK, CW)
    out_flat = _sc_scatter(idx3)
    return out_flat.reshape(BATCH, DIM, DIM)
